# baseline (speedup 1.0000x reference)
import functools

import jax
import jax.numpy as jnp
from jax import lax
from jax.experimental import pallas as pl
from jax.experimental.pallas import tpu as pltpu
from jax.experimental.pallas import tpu_sc as plsc

_B, _N, _IN_DIM, _DIM, _K = 8, 2048, 64, 64, 16
_TW = 128
_TQ = 512
_TQ2 = 512

_NW = 32
_CHUNK = 128
_TOTAL = _B * _N * _K
_NCHUNK = _TOTAL // _CHUNK
_CPW = _NCHUNK // _NW


def _tc1_body(pc_all_ref, x_all_ref, pcq_ref, w1_ref, b1_ref,
              idx_ref, t_ref):
    b = pl.program_id(0)
    q = pl.program_id(1)
    pc_all = pc_all_ref[0]
    pcq = pcq_ref[0]
    w1p = w1_ref[:3, :]

    @pl.when(q == 0)
    def _():
        x_all = x_all_ref[0]
        t = (jnp.dot(x_all, w1_ref[3:, :],
                     preferred_element_type=jnp.float32)
             + jnp.dot(pc_all, w1p,
                       preferred_element_type=jnp.float32)
             + b1_ref[...])
        t_ref[0] = jnp.concatenate(
            [t, jnp.zeros((_N, _TW - _DIM), jnp.float32)], axis=1)

    d1 = jnp.sum(pcq * pcq, axis=-1, keepdims=True)
    d2 = jnp.sum(pc_all * pc_all, axis=-1)[None, :]
    cross = jnp.dot(pcq, pc_all.T, preferred_element_type=jnp.float32)
    mat = d1 + d2 - 2.0 * cross

    iota = jax.lax.broadcasted_iota(jnp.int32, (_TQ, _N), 1)
    base = b * _N
    cols = []
    for _ in range(_K):
        mn = jnp.min(mat, axis=1, keepdims=True)
        cand = jnp.where(mat == mn, iota, _N)
        idx = jnp.min(cand, axis=1, keepdims=True)
        mat = jnp.where(cand == idx, jnp.inf, mat)
        cols.append(idx + base)
    idx_ref[0] = jnp.concatenate(cols, axis=1)


@functools.partial(jax.jit, static_argnames=("interpret",))
def _tc1(x, pc, W1, b1, interpret=False):
    nb = x.shape[0]
    grid = (nb, _N // _TQ)
    return pl.pallas_call(
        _tc1_body,
        grid=grid,
        in_specs=[
            pl.BlockSpec((1, _N, 3), lambda b, q: (b, 0, 0)),
            pl.BlockSpec((1, _N, _IN_DIM), lambda b, q: (b, 0, 0)),
            pl.BlockSpec((1, _TQ, 3), lambda b, q: (b, q, 0)),
            pl.BlockSpec((3 + _IN_DIM, _DIM), lambda b, q: (0, 0)),
            pl.BlockSpec((1, _DIM), lambda b, q: (0, 0)),
        ],
        out_specs=[
            pl.BlockSpec((1, _TQ, _K), lambda b, q: (b, q, 0)),
            pl.BlockSpec((1, _N, _TW), lambda b, q: (b, 0, 0)),
        ],
        out_shape=[
            jax.ShapeDtypeStruct((nb, _N, _K), jnp.int32),
            jax.ShapeDtypeStruct((nb, _N, _TW), jnp.float32),
        ],
        interpret=interpret,
    )(pc, x, pc, W1, b1.reshape(1, _DIM))


def _make_sc_gather_body(cpw):
    def _sc_gather_body(tab_hbm, idx_hbm, out_hbm, idx_v, rows_v, sem):
        wid = lax.axis_index("s") * 2 + lax.axis_index("c")
        row0 = wid * cpw
        pltpu.sync_copy(idx_hbm.at[pl.ds(row0, cpw)], idx_v)

        def body(c, carry):
            pltpu.async_copy(tab_hbm.at[idx_v.at[c]], rows_v, sem).wait()
            pltpu.sync_copy(rows_v, out_hbm.at[row0 + c])
            return carry

        lax.fori_loop(0, cpw, body, 0)

    return _sc_gather_body


def _sc_gather(tab, idx2d):
    nchunk = idx2d.shape[0]
    cpw = nchunk // _NW
    mesh = plsc.VectorSubcoreMesh(core_axis_name="c", subcore_axis_name="s")
    f = functools.partial(
        pl.kernel,
        mesh=mesh,
        out_type=jax.ShapeDtypeStruct((nchunk, _CHUNK, _TW), jnp.float32),
        scratch_types=[
            pltpu.VMEM((cpw, _CHUNK), jnp.int32),
            pltpu.VMEM((_CHUNK, _TW), jnp.float32),
            pltpu.SemaphoreType.DMA,
        ],
    )(_make_sc_gather_body(cpw))
    return f(tab, idx2d)


def _tc2_body(g_ref, pcq_ref, w1p_ref, w2_ref, b2_ref, out_ref):
    g = g_ref[0][:, :_DIM]
    corr = jnp.dot(pcq_ref[0], w1p_ref[...],
                   preferred_element_type=jnp.float32)
    corr_k = jnp.broadcast_to(corr[:, None, :], (_TQ2, _K, _DIM))
    h = jnp.maximum(g - corr_k.reshape(_TQ2 * _K, _DIM), 0.0)
    z = jnp.dot(h, w2_ref[...], preferred_element_type=jnp.float32)
    out_ref[0] = jnp.max(z.reshape(_TQ2, _K, _DIM), axis=1) + b2_ref[...]


@functools.partial(jax.jit, static_argnames=("interpret",))
def _tc2(g, pc, W1p, W2, b2, interpret=False):
    nb = g.shape[0]
    grid = (nb, _N // _TQ2)
    return pl.pallas_call(
        _tc2_body,
        grid=grid,
        in_specs=[
            pl.BlockSpec((1, _TQ2 * _K, _TW), lambda b, q: (b, q, 0)),

            pl.BlockSpec((1, _TQ2, 3), lambda b, q: (b, q, 0)),
            pl.BlockSpec((3, _DIM), lambda b, q: (0, 0)),
            pl.BlockSpec((_DIM, _DIM), lambda b, q: (0, 0)),
            pl.BlockSpec((1, _DIM), lambda b, q: (0, 0)),
        ],
        out_specs=pl.BlockSpec((1, _TQ2, _DIM), lambda b, q: (b, q, 0)),
        out_shape=jax.ShapeDtypeStruct((nb, _N, _DIM), jnp.float32),
        interpret=interpret,
    )(g, pc, W1p, W2, b2.reshape(1, _DIM))


def kernel(x, pc, W1, b1, W2, b2):
    nsplit = 4
    h = _B // nsplit
    gs = []
    for i in range(nsplit):
        sl = slice(i * h, (i + 1) * h)
        idx, tab = _tc1(x[sl], pc[sl], W1, b1)
        g = _sc_gather(tab.reshape(h * _N, _TW),
                       idx.reshape(-1, _CHUNK))
        gs.append(g.reshape(h, _N * _K, _TW))
    outs = [_tc2(gs[i], pc[i * h:(i + 1) * h], W1[:3, :], W2, b2)
            for i in range(nsplit)]
    return (jnp.concatenate(outs, axis=0), pc)


# speedup vs baseline: 1.1702x; 1.1702x over previous
import functools

import jax
import jax.numpy as jnp
from jax import lax
from jax.experimental import pallas as pl
from jax.experimental.pallas import tpu as pltpu
from jax.experimental.pallas import tpu_sc as plsc

_B, _N, _IN_DIM, _DIM, _K = 8, 2048, 64, 64, 16
_TW = 128
_TQ = 512
_TQ2 = 512

_NW = 32
_CHUNK = 128
_TOTAL = _B * _N * _K
_NCHUNK = _TOTAL // _CHUNK
_CPW = _NCHUNK // _NW


def _tc1_body(pc_all_ref, x_all_ref, pcq_ref, w1_ref, b1_ref,
              idx_ref, t_ref):
    b = pl.program_id(0)
    q = pl.program_id(1)
    pc_all = pc_all_ref[0]
    pcq = pcq_ref[0]
    w1p = w1_ref[:3, :]

    @pl.when(q == 0)
    def _():
        x_all = x_all_ref[0]
        t = (jnp.dot(x_all, w1_ref[3:, :],
                     preferred_element_type=jnp.float32)
             + jnp.dot(pc_all, w1p,
                       preferred_element_type=jnp.float32)
             + b1_ref[...])
        t_ref[0] = jnp.concatenate(
            [t, jnp.zeros((_N, _TW - _DIM), jnp.float32)], axis=1)

    d1 = jnp.sum(pcq * pcq, axis=-1, keepdims=True)
    d2 = jnp.sum(pc_all * pc_all, axis=-1)[None, :]
    cross = jnp.dot(pcq, pc_all.T, preferred_element_type=jnp.float32)
    mat = d1 + d2 - 2.0 * cross

    iota = jax.lax.broadcasted_iota(jnp.int32, (_TQ, _N), 1)
    base = b * _N
    cols = []
    for _ in range(_K):
        idx = jnp.argmin(mat, axis=1).astype(jnp.int32)[:, None]
        mat = jnp.where(iota == idx, jnp.inf, mat)
        cols.append(idx + base)
    idx_ref[0] = jnp.concatenate(cols, axis=1)


@functools.partial(jax.jit, static_argnames=("interpret",))
def _tc1(x, pc, W1, b1, interpret=False):
    nb = x.shape[0]
    grid = (nb, _N // _TQ)
    return pl.pallas_call(
        _tc1_body,
        grid=grid,
        in_specs=[
            pl.BlockSpec((1, _N, 3), lambda b, q: (b, 0, 0)),
            pl.BlockSpec((1, _N, _IN_DIM), lambda b, q: (b, 0, 0)),
            pl.BlockSpec((1, _TQ, 3), lambda b, q: (b, q, 0)),
            pl.BlockSpec((3 + _IN_DIM, _DIM), lambda b, q: (0, 0)),
            pl.BlockSpec((1, _DIM), lambda b, q: (0, 0)),
        ],
        out_specs=[
            pl.BlockSpec((1, _TQ, _K), lambda b, q: (b, q, 0)),
            pl.BlockSpec((1, _N, _TW), lambda b, q: (b, 0, 0)),
        ],
        out_shape=[
            jax.ShapeDtypeStruct((nb, _N, _K), jnp.int32),
            jax.ShapeDtypeStruct((nb, _N, _TW), jnp.float32),
        ],
        interpret=interpret,
    )(pc, x, pc, W1, b1.reshape(1, _DIM))


def _make_sc_gather_body(cpw):
    def _sc_gather_body(tab_hbm, idx_hbm, out_hbm, idx_v, rows_v, sem):
        wid = lax.axis_index("s") * 2 + lax.axis_index("c")
        row0 = wid * cpw
        pltpu.sync_copy(idx_hbm.at[pl.ds(row0, cpw)], idx_v)

        def body(c, carry):
            pltpu.async_copy(tab_hbm.at[idx_v.at[c]], rows_v, sem).wait()
            pltpu.sync_copy(rows_v, out_hbm.at[row0 + c])
            return carry

        lax.fori_loop(0, cpw, body, 0)

    return _sc_gather_body


def _sc_gather(tab, idx2d):
    nchunk = idx2d.shape[0]
    cpw = nchunk // _NW
    mesh = plsc.VectorSubcoreMesh(core_axis_name="c", subcore_axis_name="s")
    f = functools.partial(
        pl.kernel,
        mesh=mesh,
        out_type=jax.ShapeDtypeStruct((nchunk, _CHUNK, _TW), jnp.float32),
        scratch_types=[
            pltpu.VMEM((cpw, _CHUNK), jnp.int32),
            pltpu.VMEM((_CHUNK, _TW), jnp.float32),
            pltpu.SemaphoreType.DMA,
        ],
    )(_make_sc_gather_body(cpw))
    return f(tab, idx2d)


def _tc2_body(g_ref, pcq_ref, w1p_ref, w2_ref, b2_ref, out_ref):
    g = g_ref[0][:, :_DIM]
    corr = jnp.dot(pcq_ref[0], w1p_ref[...],
                   preferred_element_type=jnp.float32)
    corr_k = jnp.broadcast_to(corr[:, None, :], (_TQ2, _K, _DIM))
    h = jnp.maximum(g - corr_k.reshape(_TQ2 * _K, _DIM), 0.0)
    z = jnp.dot(h, w2_ref[...], preferred_element_type=jnp.float32)
    out_ref[0] = jnp.max(z.reshape(_TQ2, _K, _DIM), axis=1) + b2_ref[...]


@functools.partial(jax.jit, static_argnames=("interpret",))
def _tc2(g, pc, W1p, W2, b2, interpret=False):
    nb = g.shape[0]
    grid = (nb, _N // _TQ2)
    return pl.pallas_call(
        _tc2_body,
        grid=grid,
        in_specs=[
            pl.BlockSpec((1, _TQ2 * _K, _TW), lambda b, q: (b, q, 0)),

            pl.BlockSpec((1, _TQ2, 3), lambda b, q: (b, q, 0)),
            pl.BlockSpec((3, _DIM), lambda b, q: (0, 0)),
            pl.BlockSpec((_DIM, _DIM), lambda b, q: (0, 0)),
            pl.BlockSpec((1, _DIM), lambda b, q: (0, 0)),
        ],
        out_specs=pl.BlockSpec((1, _TQ2, _DIM), lambda b, q: (b, q, 0)),
        out_shape=jax.ShapeDtypeStruct((nb, _N, _DIM), jnp.float32),
        interpret=interpret,
    )(g, pc, W1p, W2, b2.reshape(1, _DIM))


def kernel(x, pc, W1, b1, W2, b2):
    nsplit = 2
    h = _B // nsplit
    gs = []
    for i in range(nsplit):
        sl = slice(i * h, (i + 1) * h)
        idx, tab = _tc1(x[sl], pc[sl], W1, b1)
        g = _sc_gather(tab.reshape(h * _N, _TW),
                       idx.reshape(-1, _CHUNK))
        gs.append(g.reshape(h, _N * _K, _TW))
    outs = [_tc2(gs[i], pc[i * h:(i + 1) * h], W1[:3, :], W2, b2)
            for i in range(nsplit)]
    return (jnp.concatenate(outs, axis=0), pc)
